# Initial kernel scaffold; baseline (speedup 1.0000x reference)
import jax
import jax.numpy as jnp
from jax.experimental import pallas as pl


def kernel(logits):
    raise NotImplementedError("write your pallas kernel here")


# speedup vs baseline: 83.7379x; 83.7379x over previous
import functools

import jax
import jax.numpy as jnp
from jax import lax
from jax.experimental import pallas as pl
from jax.experimental.pallas import tpu as pltpu
from jax.experimental.pallas import tpu_sc as plsc

R = 64
N = 1_000_000
K = 50
INV_T = 1.25
CHUNK = 1024
NCHUNK = 1024
NEG_INF = float("-inf")
BIG_I = 2**30
P = 512
KPAD = 64
NFULL = N // CHUNK
TAIL = N - NFULL * CHUNK

_A_CB = 131072


def _a_body(x_ref, m_ref):
  cb = pl.program_id(1)
  lane = lax.broadcasted_iota(jnp.int32, (8, CHUNK), 1)
  outs = []
  for ji in range(_A_CB // CHUNK):
    sub = x_ref[:, ji * CHUNK:(ji + 1) * CHUNK]
    rem = N - (cb * _A_CB + ji * CHUNK)
    valid = lane < rem
    sub = jnp.where(valid, sub, NEG_INF)
    outs.append(jnp.max(sub, axis=1).reshape(8, 1))
  m_ref[...] = jnp.concatenate(outs, axis=1)


def _chunk_maxes(x):
  return pl.pallas_call(
      _a_body,
      grid=(R // 8, 8),
      in_specs=[pl.BlockSpec((8, _A_CB), lambda rb, cb: (rb, cb))],
      out_specs=pl.BlockSpec((8, 128), lambda rb, cb: (rb, cb)),
      out_shape=jax.ShapeDtypeStruct((R, NCHUNK), jnp.float32),
  )(x)


def _sc_body(x_hbm, xtail_hbm, m_hbm, vals_hbm, idx_hbm,
             mv, cids, cmaxs, cbuf, tailbuf, bufv, bufi, topv, topi):
  info = plsc.get_sparse_core_info()
  nc = info.num_cores
  wid = lax.axis_index("s") * nc + lax.axis_index("c")
  iota = lax.iota(jnp.int32, 16)
  neg = jnp.full((16,), NEG_INF, jnp.float32)
  bigv = jnp.full((16,), BIG_I, jnp.int32)

  def vmax_s(v):
    m = v[0]
    for q in range(1, 16):
      m = jnp.maximum(m, v[q])
    return m

  def vmin_i(v):
    m = v[0]
    for q in range(1, 16):
      m = jnp.minimum(m, v[q])
    return m

  def reselect():
    def kbody(k, _):
      def sbody(i, carry):
        bv, bi, bp = carry
        v = bufv[pl.ds(i * 16, 16)]
        g = bufi[pl.ds(i * 16, 16)]
        p = iota + i * 16
        take = jnp.logical_or(v > bv, jnp.logical_and(v == bv, g < bi))
        return (jnp.where(take, v, bv), jnp.where(take, g, bi),
                jnp.where(take, p, bp))
      bv, bi, bp = lax.fori_loop(0, P // 16, sbody, (neg, bigv, bigv))
      bestv = vmax_s(bv)
      besti = vmin_i(jnp.where(bv == bestv, bi, BIG_I))
      bpos = vmin_i(jnp.where(jnp.logical_and(bv == bestv, bi == besti),
                              bp, BIG_I))
      tb = (k // 16) * 16
      sel = (iota + tb) == k
      topv[pl.ds(tb, 16)] = jnp.where(sel, bestv, topv[pl.ds(tb, 16)])
      topi[pl.ds(tb, 16)] = jnp.where(sel, besti, topi[pl.ds(tb, 16)])
      cb = (bpos // 16) * 16
      bufv[pl.ds(cb, 16)] = jnp.where((iota + cb) == bpos, NEG_INF,
                                      bufv[pl.ds(cb, 16)])
      return 0
    lax.fori_loop(0, K, kbody, 0)

    def wbody(i, _):
      bufv[pl.ds(i * 16, 16)] = topv[pl.ds(i * 16, 16)]
      bufi[pl.ds(i * 16, 16)] = topi[pl.ds(i * 16, 16)]
      return 0
    lax.fori_loop(0, KPAD // 16, wbody, 0)
    return jnp.int32(K), topv[pl.ds(48, 16)][K - 1 - 48]

  def do_row(r):
    rb8 = pl.multiple_of((r // 8) * 8, 8)
    rsub = r - rb8
    pltpu.sync_copy(m_hbm.at[r, 0], mv)

    def prefill(i, _):
      bufv[pl.ds(i * 16, 16)] = neg
      bufi[pl.ds(i * 16, 16)] = bigv
      return 0
    lax.fori_loop(0, P // 16, prefill, 0)

    def prefill_top(i, _):
      topv[pl.ds(i * 16, 16)] = neg
      topi[pl.ds(i * 16, 16)] = bigv
      return 0
    lax.fori_loop(0, KPAD // 16, prefill_top, 0)

    def kbody(k, _):
      def sbody(i, carry):
        bv, bi = carry
        v = mv[pl.ds(i * 16, 16)]
        g = iota + i * 16
        take = jnp.logical_or(v > bv, jnp.logical_and(v == bv, g < bi))
        return jnp.where(take, v, bv), jnp.where(take, g, bi)
      bv, bi = lax.fori_loop(0, NCHUNK // 16, sbody, (neg, bigv))
      bestv = vmax_s(bv)
      besti = vmin_i(jnp.where(bv == bestv, bi, BIG_I))
      cids[k] = besti
      cmaxs[k] = bestv
      mb = (besti // 16) * 16
      mv[pl.ds(mb, 16)] = jnp.where((iota + mb) == besti, NEG_INF,
                                    mv[pl.ds(mb, 16)])
      return 0
    lax.fori_loop(0, K, kbody, 0)

    def jbody(j, carry):
      cnt, t, done = carry
      cm = cmaxs[j]
      stop = jnp.logical_or(done, cm <= t)

      def filter_chunk(load_fn, lo, hi, cnt, t):
        def ibody(i, icarry):
          cnt, t = icarry
          v = load_fn(i)
          g = iota + (lo + i * 16)
          pm = jnp.logical_and(g < hi, v > t)
          n = plsc.all_reduce_population_count(pm)[0]
          plsc.store_compressed(bufv.at[pl.ds(cnt, 16)], v, mask=pm)
          plsc.store_compressed(bufi.at[pl.ds(cnt, 16)], g, mask=pm)
          cnt = cnt + n
          return lax.cond(cnt > P - 16, reselect, lambda: (cnt, t))
        return lax.fori_loop(0, CHUNK // 16, ibody, (cnt, t))

      def active():
        cid = cids[j]
        lo = cid * CHUNK
        hi = jnp.minimum(lo + CHUNK, N)

        def full_path():
          pltpu.sync_copy(x_hbm.at[pl.ds(rb8, 8), pl.ds(lo, CHUNK)], cbuf)
          return filter_chunk(lambda i: cbuf[rsub, pl.ds(i * 16, 16)],
                              lo, hi, cnt, t)

        def tail_path():
          pltpu.sync_copy(xtail_hbm.at[r, 0], tailbuf)
          return filter_chunk(lambda i: tailbuf[pl.ds(i * 16, 16)],
                              lo, hi, cnt, t)

        return lax.cond(cid == NFULL, tail_path, full_path)

      cnt, t = lax.cond(stop, lambda: (cnt, t), active)
      return cnt, t, stop

    lax.fori_loop(0, K, jbody,
                  (jnp.int32(0), jnp.float32(NEG_INF), False))
    reselect()
    pltpu.sync_copy(topv, vals_hbm.at[r, 0])
    pltpu.sync_copy(topi, idx_hbm.at[r, 0])

  do_row(2 * wid)
  do_row(2 * wid + 1)


def _sc_topk(x, m):
  mesh = plsc.VectorSubcoreMesh(core_axis_name="c", subcore_axis_name="s")
  fn = pl.kernel(
      _sc_body,
      out_type=[jax.ShapeDtypeStruct((R, 1, KPAD), jnp.float32),
                jax.ShapeDtypeStruct((R, 1, KPAD), jnp.int32)],
      mesh=mesh,
      compiler_params=pltpu.CompilerParams(needs_layout_passes=False),
      scratch_types=[
          pltpu.VMEM((NCHUNK,), jnp.float32),
          pltpu.SMEM((KPAD,), jnp.int32),
          pltpu.SMEM((KPAD,), jnp.float32),
          pltpu.VMEM((8, CHUNK), jnp.float32),
          pltpu.VMEM((CHUNK,), jnp.float32),
          pltpu.VMEM((P,), jnp.float32),
          pltpu.VMEM((P,), jnp.int32),
          pltpu.VMEM((KPAD,), jnp.float32),
          pltpu.VMEM((KPAD,), jnp.int32),
      ],
  )
  xtail = jnp.pad(x[:, NFULL * CHUNK:], ((0, 0), (0, CHUNK - TAIL)),
                  constant_values=NEG_INF)
  vals3, idx3 = fn(x, xtail.reshape(R, 1, CHUNK), m.reshape(R, 1, NCHUNK))
  return vals3.reshape(R, KPAD), idx3.reshape(R, KPAD)


def _c1_body(x_ref, t_ref, mt_ref, s_ref):
  cb = pl.program_id(1)

  @pl.when(cb == 0)
  def _():
    s_ref[...] = jnp.zeros_like(s_ref)

  lane = lax.broadcasted_iota(jnp.int32, (8, _A_CB), 1)
  valid = lane < (N - cb * _A_CB)
  x = x_ref[...]
  t = t_ref[...]
  mt = mt_ref[...]
  q = x * INV_T - mt
  cond = jnp.logical_and(valid, x >= t)
  e = jnp.exp(jnp.where(cond, q, NEG_INF))
  s_ref[...] += jnp.sum(e, axis=1).reshape(8, 1)


def _masked_sum(x, t, mt):
  return pl.pallas_call(
      _c1_body,
      grid=(R // 8, 8),
      in_specs=[
          pl.BlockSpec((8, _A_CB), lambda rb, cb: (rb, cb)),
          pl.BlockSpec((8, 1), lambda rb, cb: (rb, 0)),
          pl.BlockSpec((8, 1), lambda rb, cb: (rb, 0)),
      ],
      out_specs=pl.BlockSpec((8, 1), lambda rb, cb: (rb, 0)),
      out_shape=jax.ShapeDtypeStruct((R, 1), jnp.float32),
  )(x, t, mt)


_C2_CB = 32768


def _c2_body(x_ref, sc_ref, o_ref):
  x = x_ref[...]
  t = sc_ref[:, 0:1]
  mt = sc_ref[:, 1:2]
  inv_s = sc_ref[:, 2:3]
  e = jnp.exp(x * INV_T - mt) * inv_s
  o_ref[...] = jnp.where(x >= t, e, 0.0)


def _probs(x, sc):
  return pl.pallas_call(
      _c2_body,
      grid=(R // 8, N // _C2_CB + 1),
      in_specs=[
          pl.BlockSpec((8, _C2_CB), lambda rb, cb: (rb, cb)),
          pl.BlockSpec((8, 4), lambda rb, cb: (rb, 0)),
      ],
      out_specs=pl.BlockSpec((8, _C2_CB), lambda rb, cb: (rb, cb)),
      out_shape=jax.ShapeDtypeStruct((R, N), jnp.float32),
  )(x, sc)


def kernel(logits):
  m = _chunk_maxes(logits)
  vals, idxs = _sc_topk(logits, m)
  t = vals[:, K - 1:K]
  mt = vals[:, 0:1] * INV_T
  s = _masked_sum(logits, t, mt)
  sc = jnp.concatenate([t, mt, 1.0 / s, jnp.zeros_like(s)], axis=1)
  probs = _probs(logits, sc)
  return probs, idxs[:, :K]


# speedup vs baseline: 103.7873x; 1.2394x over previous
import functools

import jax
import jax.numpy as jnp
from jax import lax
from jax.experimental import pallas as pl
from jax.experimental.pallas import tpu as pltpu
from jax.experimental.pallas import tpu_sc as plsc

R = 64
N = 1_000_000
K = 50
INV_T = 1.25
CHUNK = 1024
NCHUNK = 1024
NEG_INF = float("-inf")
BIG_I = 2**30
P = 1280
KPAD = 64
NFULL = N // CHUNK
TAIL = N - NFULL * CHUNK

_A_CB = 131072


def _a_body(x_ref, m_ref):
  cb = pl.program_id(1)
  lane = lax.broadcasted_iota(jnp.int32, (8, CHUNK), 1)
  outs = []
  for ji in range(_A_CB // CHUNK):
    sub = x_ref[:, ji * CHUNK:(ji + 1) * CHUNK]
    rem = N - (cb * _A_CB + ji * CHUNK)
    valid = lane < rem
    sub = jnp.where(valid, sub, NEG_INF)
    outs.append(jnp.max(sub, axis=1).reshape(8, 1))
  m_ref[...] = jnp.concatenate(outs, axis=1)


def _chunk_maxes(x):
  return pl.pallas_call(
      _a_body,
      grid=(R // 8, 8),
      in_specs=[pl.BlockSpec((8, _A_CB), lambda rb, cb: (rb, cb))],
      out_specs=pl.BlockSpec((8, 128), lambda rb, cb: (rb, cb)),
      out_shape=jax.ShapeDtypeStruct((R, NCHUNK), jnp.float32),
  )(x)


def _sc_body(x_hbm, xtail_hbm, m_hbm, vals_hbm, idx_hbm,
             mv, cids, cmaxs, cbuf, tailbuf, bufv, bufi, topv, topi):
  info = plsc.get_sparse_core_info()
  nc = info.num_cores
  wid = lax.axis_index("s") * nc + lax.axis_index("c")
  iota = lax.iota(jnp.int32, 16)
  neg = jnp.full((16,), NEG_INF, jnp.float32)
  bigv = jnp.full((16,), BIG_I, jnp.int32)

  def vmax_s(v):
    m = v[0]
    for q in range(1, 16):
      m = jnp.maximum(m, v[q])
    return m

  def vmin_i(v):
    m = v[0]
    for q in range(1, 16):
      m = jnp.minimum(m, v[q])
    return m

  def reselect(cnt):
    nsl = (cnt + 15) // 16
    def kbody(k, _):
      def sbody(i, carry):
        bv, bi, bp = carry
        p = iota + i * 16
        live = p < cnt
        v = jnp.where(live, bufv[pl.ds(i * 16, 16)], NEG_INF)
        g = jnp.where(live, bufi[pl.ds(i * 16, 16)], BIG_I)
        take = jnp.logical_or(v > bv, jnp.logical_and(v == bv, g < bi))
        return (jnp.where(take, v, bv), jnp.where(take, g, bi),
                jnp.where(take, p, bp))
      bv, bi, bp = lax.fori_loop(0, nsl, sbody, (neg, bigv, bigv))
      bestv = vmax_s(bv)
      besti = vmin_i(jnp.where(bv == bestv, bi, BIG_I))
      bpos = vmin_i(jnp.where(jnp.logical_and(bv == bestv, bi == besti),
                              bp, BIG_I))
      tb = (k // 16) * 16
      sel = (iota + tb) == k
      topv[pl.ds(tb, 16)] = jnp.where(sel, bestv, topv[pl.ds(tb, 16)])
      topi[pl.ds(tb, 16)] = jnp.where(sel, besti, topi[pl.ds(tb, 16)])
      cb = (bpos // 16) * 16
      bufv[pl.ds(cb, 16)] = jnp.where((iota + cb) == bpos, NEG_INF,
                                      bufv[pl.ds(cb, 16)])
      return 0
    lax.fori_loop(0, K, kbody, 0)

    def wbody(i, _):
      bufv[pl.ds(i * 16, 16)] = topv[pl.ds(i * 16, 16)]
      bufi[pl.ds(i * 16, 16)] = topi[pl.ds(i * 16, 16)]
      return 0
    lax.fori_loop(0, KPAD // 16, wbody, 0)
    return jnp.int32(K), topv[pl.ds(48, 16)][K - 1 - 48]

  def do_row(r):
    rb8 = pl.multiple_of((r // 8) * 8, 8)
    rsub = r - rb8
    pltpu.sync_copy(m_hbm.at[r, 0], mv)

    def prefill(i, _):
      bufv[pl.ds(i * 16, 16)] = neg
      bufi[pl.ds(i * 16, 16)] = bigv
      return 0
    lax.fori_loop(0, P // 16, prefill, 0)

    def prefill_top(i, _):
      topv[pl.ds(i * 16, 16)] = neg
      topi[pl.ds(i * 16, 16)] = bigv
      return 0
    lax.fori_loop(0, KPAD // 16, prefill_top, 0)

    def kbody(k, _):
      def sbody(i, carry):
        bv, bi = carry
        v = mv[pl.ds(i * 16, 16)]
        g = iota + i * 16
        take = jnp.logical_or(v > bv, jnp.logical_and(v == bv, g < bi))
        return jnp.where(take, v, bv), jnp.where(take, g, bi)
      bv, bi = lax.fori_loop(0, NCHUNK // 16, sbody, (neg, bigv))
      bestv = vmax_s(bv)
      besti = vmin_i(jnp.where(bv == bestv, bi, BIG_I))
      cids[k] = besti
      cmaxs[k] = bestv
      mb = (besti // 16) * 16
      mv[pl.ds(mb, 16)] = jnp.where((iota + mb) == besti, NEG_INF,
                                    mv[pl.ds(mb, 16)])
      return 0
    lax.fori_loop(0, K, kbody, 0)

    def jbody(j, carry):
      cnt, t, done = carry
      cm = cmaxs[j]
      stop = jnp.logical_or(done, cm <= t)

      def filter_chunk(load_fn, lo, hi, cnt, t):
        def ibody(i, cnt):
          v = load_fn(i)
          g = iota + (lo + i * 16)
          pm = jnp.logical_and(g < hi, v > t)
          n = plsc.all_reduce_population_count(pm)[0]
          plsc.store_compressed(bufv.at[pl.ds(cnt, 16)], v, mask=pm)
          plsc.store_compressed(bufi.at[pl.ds(cnt, 16)], g, mask=pm)
          return cnt + n
        cnt = lax.fori_loop(0, CHUNK // 16, ibody, cnt, unroll=4)
        return cnt, t

      def active():
        cnt2, t2 = lax.cond(cnt > P - CHUNK, lambda: reselect(cnt),
                            lambda: (cnt, t))
        cid = cids[j]
        lo = cid * CHUNK
        hi = jnp.minimum(lo + CHUNK, N)

        def full_path():
          pltpu.sync_copy(x_hbm.at[pl.ds(rb8, 8), pl.ds(lo, CHUNK)], cbuf)
          return filter_chunk(lambda i: cbuf[rsub, pl.ds(i * 16, 16)],
                              lo, hi, cnt2, t2)

        def tail_path():
          pltpu.sync_copy(xtail_hbm.at[r, 0], tailbuf)
          return filter_chunk(lambda i: tailbuf[pl.ds(i * 16, 16)],
                              lo, hi, cnt2, t2)

        return lax.cond(cid == NFULL, tail_path, full_path)

      cnt, t = lax.cond(stop, lambda: (cnt, t), active)
      return cnt, t, stop

    t0s = jnp.full((16,), cmaxs[K - 1], jnp.float32)
    b = plsc.bitcast(t0s, jnp.uint32)
    sgn = b >> jnp.uint32(31)
    key = b ^ jnp.where(sgn == jnp.uint32(1),
                        jnp.uint32(0xFFFFFFFF), jnp.uint32(0x80000000))
    key = key - jnp.uint32(1)
    sgn2 = key >> jnp.uint32(31)
    b2 = key ^ jnp.where(sgn2 == jnp.uint32(1),
                         jnp.uint32(0x80000000), jnp.uint32(0xFFFFFFFF))
    t0m = plsc.bitcast(b2, jnp.float32)[0]

    cnt_f, _, _ = lax.fori_loop(0, K, jbody, (jnp.int32(0), t0m, False))
    reselect(cnt_f)
    pltpu.sync_copy(topv, vals_hbm.at[r, 0])
    pltpu.sync_copy(topi, idx_hbm.at[r, 0])

  do_row(2 * wid)
  do_row(2 * wid + 1)


def _sc_topk(x, m):
  mesh = plsc.VectorSubcoreMesh(core_axis_name="c", subcore_axis_name="s")
  fn = pl.kernel(
      _sc_body,
      out_type=[jax.ShapeDtypeStruct((R, 1, KPAD), jnp.float32),
                jax.ShapeDtypeStruct((R, 1, KPAD), jnp.int32)],
      mesh=mesh,
      compiler_params=pltpu.CompilerParams(needs_layout_passes=False),
      scratch_types=[
          pltpu.VMEM((NCHUNK,), jnp.float32),
          pltpu.SMEM((KPAD,), jnp.int32),
          pltpu.SMEM((KPAD,), jnp.float32),
          pltpu.VMEM((8, CHUNK), jnp.float32),
          pltpu.VMEM((CHUNK,), jnp.float32),
          pltpu.VMEM((P,), jnp.float32),
          pltpu.VMEM((P,), jnp.int32),
          pltpu.VMEM((KPAD,), jnp.float32),
          pltpu.VMEM((KPAD,), jnp.int32),
      ],
  )
  xtail = jnp.pad(x[:, NFULL * CHUNK:], ((0, 0), (0, CHUNK - TAIL)),
                  constant_values=NEG_INF)
  vals3, idx3 = fn(x, xtail.reshape(R, 1, CHUNK), m.reshape(R, 1, NCHUNK))
  return vals3.reshape(R, KPAD), idx3.reshape(R, KPAD)


def _c1_body(x_ref, t_ref, mt_ref, s_ref):
  cb = pl.program_id(1)

  @pl.when(cb == 0)
  def _():
    s_ref[...] = jnp.zeros_like(s_ref)

  lane = lax.broadcasted_iota(jnp.int32, (8, _A_CB), 1)
  valid = lane < (N - cb * _A_CB)
  x = x_ref[...]
  t = t_ref[...]
  mt = mt_ref[...]
  q = x * INV_T - mt
  cond = jnp.logical_and(valid, x >= t)
  e = jnp.exp(jnp.where(cond, q, NEG_INF))
  s_ref[...] += jnp.sum(e, axis=1).reshape(8, 1)


def _masked_sum(x, t, mt):
  return pl.pallas_call(
      _c1_body,
      grid=(R // 8, 8),
      in_specs=[
          pl.BlockSpec((8, _A_CB), lambda rb, cb: (rb, cb)),
          pl.BlockSpec((8, 1), lambda rb, cb: (rb, 0)),
          pl.BlockSpec((8, 1), lambda rb, cb: (rb, 0)),
      ],
      out_specs=pl.BlockSpec((8, 1), lambda rb, cb: (rb, 0)),
      out_shape=jax.ShapeDtypeStruct((R, 1), jnp.float32),
  )(x, t, mt)


_C2_CB = 32768


def _c2_body(x_ref, sc_ref, o_ref):
  x = x_ref[...]
  t = sc_ref[:, 0:1]
  mt = sc_ref[:, 1:2]
  inv_s = sc_ref[:, 2:3]
  e = jnp.exp(x * INV_T - mt) * inv_s
  o_ref[...] = jnp.where(x >= t, e, 0.0)


def _probs(x, sc):
  return pl.pallas_call(
      _c2_body,
      grid=(R // 8, N // _C2_CB + 1),
      in_specs=[
          pl.BlockSpec((8, _C2_CB), lambda rb, cb: (rb, cb)),
          pl.BlockSpec((8, 4), lambda rb, cb: (rb, 0)),
      ],
      out_specs=pl.BlockSpec((8, _C2_CB), lambda rb, cb: (rb, cb)),
      out_shape=jax.ShapeDtypeStruct((R, N), jnp.float32),
  )(x, sc)


def kernel(logits):
  m = _chunk_maxes(logits)
  vals, idxs = _sc_topk(logits, m)
  t = vals[:, K - 1:K]
  mt = vals[:, 0:1] * INV_T
  s = _masked_sum(logits, t, mt)
  sc = jnp.concatenate([t, mt, 1.0 / s, jnp.zeros_like(s)], axis=1)
  probs = _probs(logits, sc)
  return probs, idxs[:, :K]


# speedup vs baseline: 104.1133x; 1.0031x over previous
import functools

import jax
import jax.numpy as jnp
from jax import lax
from jax.experimental import pallas as pl
from jax.experimental.pallas import tpu as pltpu
from jax.experimental.pallas import tpu_sc as plsc

R = 64
N = 1_000_000
K = 50
INV_T = 1.25
CHUNK = 1024
NCHUNK = 1024
NEG_INF = float("-inf")
BIG_I = 2**30
P = 1280
KPAD = 64
NFULL = N // CHUNK
TAIL = N - NFULL * CHUNK

_A_CB = 131072


def _a_body(x_ref, m_ref):
  cb = pl.program_id(1)

  @pl.when(cb < 7)
  def _():
    outs = []
    for ji in range(_A_CB // CHUNK):
      sub = x_ref[:, ji * CHUNK:(ji + 1) * CHUNK]
      outs.append(jnp.max(sub, axis=1).reshape(8, 1))
    m_ref[...] = jnp.concatenate(outs, axis=1)

  @pl.when(cb == 7)
  def _():
    lane = lax.broadcasted_iota(jnp.int32, (8, CHUNK), 1)
    outs = []
    for ji in range(_A_CB // CHUNK):
      sub = x_ref[:, ji * CHUNK:(ji + 1) * CHUNK]
      rem = N - (7 * _A_CB + ji * CHUNK)
      sub = jnp.where(lane < rem, sub, NEG_INF)
      outs.append(jnp.max(sub, axis=1).reshape(8, 1))
    m_ref[...] = jnp.concatenate(outs, axis=1)


def _chunk_maxes(x):
  return pl.pallas_call(
      _a_body,
      grid=(R // 8, 8),
      in_specs=[pl.BlockSpec((8, _A_CB), lambda rb, cb: (rb, cb))],
      out_specs=pl.BlockSpec((8, 128), lambda rb, cb: (rb, cb)),
      out_shape=jax.ShapeDtypeStruct((R, NCHUNK), jnp.float32),
  )(x)


def _sc_body(x_hbm, xtail_hbm, m_hbm, vals_hbm, idx_hbm,
             mv, cids, cmaxs, cbuf, tailbuf, bufv, bufi, topv, topi):
  info = plsc.get_sparse_core_info()
  nc = info.num_cores
  wid = lax.axis_index("s") * nc + lax.axis_index("c")
  iota = lax.iota(jnp.int32, 16)
  neg = jnp.full((16,), NEG_INF, jnp.float32)
  bigv = jnp.full((16,), BIG_I, jnp.int32)

  def vmax_s(v):
    m = v[0]
    for q in range(1, 16):
      m = jnp.maximum(m, v[q])
    return m

  def vmin_i(v):
    m = v[0]
    for q in range(1, 16):
      m = jnp.minimum(m, v[q])
    return m

  def reselect(cnt):
    nsl = (cnt + 15) // 16
    def kbody(k, _):
      def sbody(i, carry):
        bv, bi, bp = carry
        p = iota + i * 16
        live = p < cnt
        v = jnp.where(live, bufv[pl.ds(i * 16, 16)], NEG_INF)
        g = jnp.where(live, bufi[pl.ds(i * 16, 16)], BIG_I)
        take = jnp.logical_or(v > bv, jnp.logical_and(v == bv, g < bi))
        return (jnp.where(take, v, bv), jnp.where(take, g, bi),
                jnp.where(take, p, bp))
      bv, bi, bp = lax.fori_loop(0, nsl, sbody, (neg, bigv, bigv))
      bestv = vmax_s(bv)
      besti = vmin_i(jnp.where(bv == bestv, bi, BIG_I))
      bpos = vmin_i(jnp.where(jnp.logical_and(bv == bestv, bi == besti),
                              bp, BIG_I))
      tb = (k // 16) * 16
      sel = (iota + tb) == k
      topv[pl.ds(tb, 16)] = jnp.where(sel, bestv, topv[pl.ds(tb, 16)])
      topi[pl.ds(tb, 16)] = jnp.where(sel, besti, topi[pl.ds(tb, 16)])
      cb = (bpos // 16) * 16
      bufv[pl.ds(cb, 16)] = jnp.where((iota + cb) == bpos, NEG_INF,
                                      bufv[pl.ds(cb, 16)])
      return 0
    lax.fori_loop(0, K, kbody, 0)

    def wbody(i, _):
      bufv[pl.ds(i * 16, 16)] = topv[pl.ds(i * 16, 16)]
      bufi[pl.ds(i * 16, 16)] = topi[pl.ds(i * 16, 16)]
      return 0
    lax.fori_loop(0, KPAD // 16, wbody, 0)
    return jnp.int32(K), topv[pl.ds(48, 16)][K - 1 - 48]

  def do_row(r):
    rb8 = pl.multiple_of((r // 8) * 8, 8)
    rsub = r - rb8
    pltpu.sync_copy(m_hbm.at[r, 0], mv)

    def prefill(i, _):
      bufv[pl.ds(i * 16, 16)] = neg
      bufi[pl.ds(i * 16, 16)] = bigv
      return 0
    lax.fori_loop(0, P // 16, prefill, 0)

    def prefill_top(i, _):
      topv[pl.ds(i * 16, 16)] = neg
      topi[pl.ds(i * 16, 16)] = bigv
      return 0
    lax.fori_loop(0, KPAD // 16, prefill_top, 0)

    def kbody(k, _):
      def sbody(i, carry):
        bv, bi = carry
        v = mv[pl.ds(i * 16, 16)]
        g = iota + i * 16
        take = jnp.logical_or(v > bv, jnp.logical_and(v == bv, g < bi))
        return jnp.where(take, v, bv), jnp.where(take, g, bi)
      bv, bi = lax.fori_loop(0, NCHUNK // 16, sbody, (neg, bigv))
      bestv = vmax_s(bv)
      besti = vmin_i(jnp.where(bv == bestv, bi, BIG_I))
      cids[k] = besti
      cmaxs[k] = bestv
      mb = (besti // 16) * 16
      mv[pl.ds(mb, 16)] = jnp.where((iota + mb) == besti, NEG_INF,
                                    mv[pl.ds(mb, 16)])
      return 0
    lax.fori_loop(0, K, kbody, 0)

    def jbody(j, carry):
      cnt, t, done = carry
      cm = cmaxs[j]
      stop = jnp.logical_or(done, cm <= t)

      def filter_chunk(load_fn, lo, hi, cnt, t):
        def ibody(i, cnt):
          v = load_fn(i)
          g = iota + (lo + i * 16)
          pm = jnp.logical_and(g < hi, v > t)
          n = plsc.all_reduce_population_count(pm)[0]
          plsc.store_compressed(bufv.at[pl.ds(cnt, 16)], v, mask=pm)
          plsc.store_compressed(bufi.at[pl.ds(cnt, 16)], g, mask=pm)
          return cnt + n
        cnt = lax.fori_loop(0, CHUNK // 16, ibody, cnt, unroll=4)
        return cnt, t

      def active():
        cnt2, t2 = lax.cond(cnt > P - CHUNK, lambda: reselect(cnt),
                            lambda: (cnt, t))
        cid = cids[j]
        lo = cid * CHUNK
        hi = jnp.minimum(lo + CHUNK, N)

        def full_path():
          pltpu.sync_copy(x_hbm.at[pl.ds(rb8, 8), pl.ds(lo, CHUNK)], cbuf)
          return filter_chunk(lambda i: cbuf[rsub, pl.ds(i * 16, 16)],
                              lo, hi, cnt2, t2)

        def tail_path():
          pltpu.sync_copy(xtail_hbm.at[r, 0], tailbuf)
          return filter_chunk(lambda i: tailbuf[pl.ds(i * 16, 16)],
                              lo, hi, cnt2, t2)

        return lax.cond(cid == NFULL, tail_path, full_path)

      cnt, t = lax.cond(stop, lambda: (cnt, t), active)
      return cnt, t, stop

    t0s = jnp.full((16,), cmaxs[K - 1], jnp.float32)
    b = plsc.bitcast(t0s, jnp.uint32)
    sgn = b >> jnp.uint32(31)
    key = b ^ jnp.where(sgn == jnp.uint32(1),
                        jnp.uint32(0xFFFFFFFF), jnp.uint32(0x80000000))
    key = key - jnp.uint32(1)
    sgn2 = key >> jnp.uint32(31)
    b2 = key ^ jnp.where(sgn2 == jnp.uint32(1),
                         jnp.uint32(0x80000000), jnp.uint32(0xFFFFFFFF))
    t0m = plsc.bitcast(b2, jnp.float32)[0]

    cnt_f, _, _ = lax.fori_loop(0, K, jbody, (jnp.int32(0), t0m, False))
    reselect(cnt_f)
    pltpu.sync_copy(topv, vals_hbm.at[r, 0])
    pltpu.sync_copy(topi, idx_hbm.at[r, 0])

  do_row(2 * wid)
  do_row(2 * wid + 1)


def _sc_topk(x, m):
  mesh = plsc.VectorSubcoreMesh(core_axis_name="c", subcore_axis_name="s")
  fn = pl.kernel(
      _sc_body,
      out_type=[jax.ShapeDtypeStruct((R, 1, KPAD), jnp.float32),
                jax.ShapeDtypeStruct((R, 1, KPAD), jnp.int32)],
      mesh=mesh,
      compiler_params=pltpu.CompilerParams(needs_layout_passes=False),
      scratch_types=[
          pltpu.VMEM((NCHUNK,), jnp.float32),
          pltpu.SMEM((KPAD,), jnp.int32),
          pltpu.SMEM((KPAD,), jnp.float32),
          pltpu.VMEM((8, CHUNK), jnp.float32),
          pltpu.VMEM((CHUNK,), jnp.float32),
          pltpu.VMEM((P,), jnp.float32),
          pltpu.VMEM((P,), jnp.int32),
          pltpu.VMEM((KPAD,), jnp.float32),
          pltpu.VMEM((KPAD,), jnp.int32),
      ],
  )
  xtail = jnp.pad(x[:, NFULL * CHUNK:], ((0, 0), (0, CHUNK - TAIL)),
                  constant_values=NEG_INF)
  vals3, idx3 = fn(x, xtail.reshape(R, 1, CHUNK), m.reshape(R, 1, NCHUNK))
  return vals3.reshape(R, KPAD), idx3.reshape(R, KPAD)


def _c1_body(x_ref, t_ref, c_ref):
  cb = pl.program_id(1)

  @pl.when(cb == 0)
  def _():
    c_ref[...] = jnp.zeros_like(c_ref)

  x = x_ref[...]
  t = t_ref[...]
  eq = (x == t).astype(jnp.float32)

  @pl.when(cb < 7)
  def _():
    c_ref[...] += jnp.sum(eq, axis=1).reshape(8, 1)

  @pl.when(cb == 7)
  def _():
    lane = lax.broadcasted_iota(jnp.int32, (8, _A_CB), 1)
    valid = lane < (N - 7 * _A_CB)
    c_ref[...] += jnp.sum(jnp.where(valid, eq, 0.0), axis=1).reshape(8, 1)


def _tie_count(x, t):
  return pl.pallas_call(
      _c1_body,
      grid=(R // 8, 8),
      in_specs=[
          pl.BlockSpec((8, _A_CB), lambda rb, cb: (rb, cb)),
          pl.BlockSpec((8, 1), lambda rb, cb: (rb, 0)),
      ],
      out_specs=pl.BlockSpec((8, 1), lambda rb, cb: (rb, 0)),
      out_shape=jax.ShapeDtypeStruct((R, 1), jnp.float32),
  )(x, t)


_C2_CB = 32768


def _c2_body(x_ref, sc_ref, o_ref):
  x = x_ref[...]
  t = sc_ref[:, 0:1]
  mt = sc_ref[:, 1:2]
  inv_s = sc_ref[:, 2:3]
  e = jnp.exp(x * INV_T - mt) * inv_s
  o_ref[...] = jnp.where(x >= t, e, 0.0)


def _probs(x, sc):
  return pl.pallas_call(
      _c2_body,
      grid=(R // 8, N // _C2_CB + 1),
      in_specs=[
          pl.BlockSpec((8, _C2_CB), lambda rb, cb: (rb, cb)),
          pl.BlockSpec((8, 4), lambda rb, cb: (rb, 0)),
      ],
      out_specs=pl.BlockSpec((8, _C2_CB), lambda rb, cb: (rb, cb)),
      out_shape=jax.ShapeDtypeStruct((R, N), jnp.float32),
  )(x, sc)


def kernel(logits):
  m = _chunk_maxes(logits)
  vals, idxs = _sc_topk(logits, m)
  t = vals[:, K - 1:K]
  mt = vals[:, 0:1] * INV_T
  c_total = _tie_count(logits, t)
  topk = vals[:, :K]
  e_top = jnp.exp(topk * INV_T - mt)
  c_in = jnp.sum((topk == t).astype(jnp.float32), axis=1, keepdims=True)
  s = jnp.sum(e_top, axis=1, keepdims=True) + (
      c_total - c_in) * jnp.exp(t * INV_T - mt)
  sc = jnp.concatenate([t, mt, 1.0 / s, jnp.zeros_like(s)], axis=1)
  probs = _probs(logits, sc)
  return probs, idxs[:, :K]


# speedup vs baseline: 116.8263x; 1.1221x over previous
import functools

import jax
import jax.numpy as jnp
from jax import lax
from jax.experimental import pallas as pl
from jax.experimental.pallas import tpu as pltpu
from jax.experimental.pallas import tpu_sc as plsc

R = 64
N = 1_000_000
K = 50
INV_T = 1.25
CHUNK = 1024
NCHUNK = 1024
NEG_INF = float("-inf")
BIG_I = 2**30
P = 1280
KPAD = 64
NFULL = N // CHUNK
TAIL = N - NFULL * CHUNK

_A_CB = 131072


def _a_body(x_ref, m_ref):
  cb = pl.program_id(1)

  @pl.when(cb < 7)
  def _():
    outs = []
    for ji in range(_A_CB // CHUNK):
      sub = x_ref[:, ji * CHUNK:(ji + 1) * CHUNK]
      outs.append(jnp.max(sub, axis=1).reshape(8, 1))
    m_ref[...] = jnp.concatenate(outs, axis=1)

  @pl.when(cb == 7)
  def _():
    lane = lax.broadcasted_iota(jnp.int32, (8, CHUNK), 1)
    outs = []
    for ji in range(_A_CB // CHUNK):
      sub = x_ref[:, ji * CHUNK:(ji + 1) * CHUNK]
      rem = N - (7 * _A_CB + ji * CHUNK)
      sub = jnp.where(lane < rem, sub, NEG_INF)
      outs.append(jnp.max(sub, axis=1).reshape(8, 1))
    m_ref[...] = jnp.concatenate(outs, axis=1)


def _chunk_maxes(x):
  return pl.pallas_call(
      _a_body,
      grid=(R // 8, 8),
      in_specs=[pl.BlockSpec((8, _A_CB), lambda rb, cb: (rb, cb))],
      out_specs=pl.BlockSpec((8, 128), lambda rb, cb: (rb, cb)),
      out_shape=jax.ShapeDtypeStruct((R, NCHUNK), jnp.float32),
  )(x)


def _sc_body(x_hbm, xtail_hbm, m_hbm, vals_hbm, idx_hbm,
             mv, cids, cmaxs, cbuf, cbuf2, tailbuf, bufv, bufi, topv, topi,
             supv, sem):
  info = plsc.get_sparse_core_info()
  nc = info.num_cores
  wid = lax.axis_index("s") * nc + lax.axis_index("c")
  iota = lax.iota(jnp.int32, 16)
  neg = jnp.full((16,), NEG_INF, jnp.float32)
  bigv = jnp.full((16,), BIG_I, jnp.int32)

  def vmax_s(v):
    m = v[0]
    for q in range(1, 16):
      m = jnp.maximum(m, v[q])
    return m

  def vmin_i(v):
    m = v[0]
    for q in range(1, 16):
      m = jnp.minimum(m, v[q])
    return m

  def reselect(cnt):
    nsl = (cnt + 15) // 16
    def kbody(k, _):
      def sbody(i, carry):
        bv, bi, bp = carry
        p = iota + i * 16
        live = p < cnt
        v = jnp.where(live, bufv[pl.ds(i * 16, 16)], NEG_INF)
        g = jnp.where(live, bufi[pl.ds(i * 16, 16)], BIG_I)
        take = jnp.logical_or(v > bv, jnp.logical_and(v == bv, g < bi))
        return (jnp.where(take, v, bv), jnp.where(take, g, bi),
                jnp.where(take, p, bp))
      bv, bi, bp = lax.fori_loop(0, nsl, sbody, (neg, bigv, bigv))
      bestv = vmax_s(bv)
      besti = vmin_i(jnp.where(bv == bestv, bi, BIG_I))
      bpos = vmin_i(jnp.where(jnp.logical_and(bv == bestv, bi == besti),
                              bp, BIG_I))
      tb = (k // 16) * 16
      sel = (iota + tb) == k
      topv[pl.ds(tb, 16)] = jnp.where(sel, bestv, topv[pl.ds(tb, 16)])
      topi[pl.ds(tb, 16)] = jnp.where(sel, besti, topi[pl.ds(tb, 16)])
      cb = (bpos // 16) * 16
      bufv[pl.ds(cb, 16)] = jnp.where((iota + cb) == bpos, NEG_INF,
                                      bufv[pl.ds(cb, 16)])
      return 0
    lax.fori_loop(0, K, kbody, 0)

    def wbody(i, _):
      bufv[pl.ds(i * 16, 16)] = topv[pl.ds(i * 16, 16)]
      bufi[pl.ds(i * 16, 16)] = topi[pl.ds(i * 16, 16)]
      return 0
    lax.fori_loop(0, KPAD // 16, wbody, 0)
    return jnp.int32(K), topv[pl.ds(48, 16)][K - 1 - 48]

  def do_row(r):
    rb8 = pl.multiple_of((r // 8) * 8, 8)
    rsub = r - rb8
    pltpu.sync_copy(m_hbm.at[r, 0], mv)

    def prefill(i, _):
      bufv[pl.ds(i * 16, 16)] = neg
      bufi[pl.ds(i * 16, 16)] = bigv
      return 0
    lax.fori_loop(0, P // 16, prefill, 0)

    def prefill_top(i, _):
      topv[pl.ds(i * 16, 16)] = neg
      topi[pl.ds(i * 16, 16)] = bigv
      return 0
    lax.fori_loop(0, KPAD // 16, prefill_top, 0)

    def initsup(b, _):
      m16 = vmax_s(mv[pl.ds(b * 16, 16)])
      sb = (b // 16) * 16
      supv[pl.ds(sb, 16)] = jnp.where((iota + sb) == b, m16,
                                      supv[pl.ds(sb, 16)])
      return 0
    lax.fori_loop(0, NCHUNK // 16, initsup, 0)

    def kbody(k, _):
      def ssup(i, carry):
        bv, bb = carry
        v = supv[pl.ds(i * 16, 16)]
        g = iota + i * 16
        take = jnp.logical_or(v > bv, jnp.logical_and(v == bv, g < bb))
        return jnp.where(take, v, bv), jnp.where(take, g, bb)
      bv, bb = lax.fori_loop(0, NCHUNK // 256, ssup, (neg, bigv))
      bestv = vmax_s(bv)
      bblock = vmin_i(jnp.where(bv == bestv, bb, BIG_I))
      sl = mv[pl.ds(bblock * 16, 16)]
      lane = vmin_i(jnp.where(sl == bestv, iota, BIG_I))
      besti = bblock * 16 + lane
      cids[k] = besti
      cmaxs[k] = bestv
      sl2 = jnp.where(iota == lane, NEG_INF, sl)
      mv[pl.ds(bblock * 16, 16)] = sl2
      m2 = vmax_s(sl2)
      sb = (bblock // 16) * 16
      supv[pl.ds(sb, 16)] = jnp.where((iota + sb) == bblock, m2,
                                      supv[pl.ds(sb, 16)])
      return 0
    lax.fori_loop(0, K, kbody, 0)

    def filter_chunk(load_fn, lo, hi, cnt, t):
      def ibody(i, cnt):
        v = load_fn(i)
        g = iota + (lo + i * 16)
        pm = jnp.logical_and(g < hi, v > t)
        n = plsc.all_reduce_population_count(pm)[0]
        plsc.store_compressed(bufv.at[pl.ds(cnt, 16)], v, mask=pm)
        plsc.store_compressed(bufi.at[pl.ds(cnt, 16)], g, mask=pm)
        return cnt + n
      cnt = lax.fori_loop(0, CHUNK // 16, ibody, cnt, unroll=4)
      return cnt, t

    def start_fetch(j, buf):
      @pl.when(cids[j] != NFULL)
      def _():
        lo = cids[j] * CHUNK
        pltpu.make_async_copy(
            x_hbm.at[pl.ds(rb8, 8), pl.ds(lo, CHUNK)], buf, sem).start()

    def wait_fetch(j, buf):
      @pl.when(cids[j] != NFULL)
      def _():
        lo = cids[j] * CHUNK
        pltpu.make_async_copy(
            x_hbm.at[pl.ds(rb8, 8), pl.ds(lo, CHUNK)], buf, sem).wait()

    def process(j, buf, cnt, t):
      cnt, t = lax.cond(cnt > P - CHUNK, lambda: reselect(cnt),
                        lambda: (cnt, t))
      cid = cids[j]
      lo = cid * CHUNK
      hi = jnp.minimum(lo + CHUNK, N)

      def full_path():
        return filter_chunk(lambda i: buf[rsub, pl.ds(i * 16, 16)],
                            lo, hi, cnt, t)

      def tail_path():
        pltpu.sync_copy(xtail_hbm.at[r, 0], tailbuf)
        return filter_chunk(lambda i: tailbuf[pl.ds(i * 16, 16)],
                            lo, hi, cnt, t)

      return lax.cond(cid == NFULL, tail_path, full_path)

    t0s = jnp.full((16,), cmaxs[K - 1], jnp.float32)
    b = plsc.bitcast(t0s, jnp.uint32)
    sgn = b >> jnp.uint32(31)
    key = b ^ jnp.where(sgn == jnp.uint32(1),
                        jnp.uint32(0xFFFFFFFF), jnp.uint32(0x80000000))
    key = key - jnp.uint32(1)
    sgn2 = key >> jnp.uint32(31)
    b2 = key ^ jnp.where(sgn2 == jnp.uint32(1),
                         jnp.uint32(0x80000000), jnp.uint32(0xFFFFFFFF))
    t0m = plsc.bitcast(b2, jnp.float32)[0]

    start_fetch(0, cbuf)

    def pairbody(p, carry):
      cnt, t = carry
      j0 = 2 * p
      wait_fetch(j0, cbuf)
      start_fetch(j0 + 1, cbuf2)
      cnt, t = process(j0, cbuf, cnt, t)
      wait_fetch(j0 + 1, cbuf2)

      @pl.when(p < K // 2 - 1)
      def _():
        start_fetch(j0 + 2, cbuf)

      return process(j0 + 1, cbuf2, cnt, t)

    cnt_f, _ = lax.fori_loop(0, K // 2, pairbody, (jnp.int32(0), t0m))
    reselect(cnt_f)
    pltpu.sync_copy(topv, vals_hbm.at[r, 0])
    pltpu.sync_copy(topi, idx_hbm.at[r, 0])

  do_row(2 * wid)
  do_row(2 * wid + 1)


def _sc_topk(x, m):
  mesh = plsc.VectorSubcoreMesh(core_axis_name="c", subcore_axis_name="s")
  fn = pl.kernel(
      _sc_body,
      out_type=[jax.ShapeDtypeStruct((R, 1, KPAD), jnp.float32),
                jax.ShapeDtypeStruct((R, 1, KPAD), jnp.int32)],
      mesh=mesh,
      compiler_params=pltpu.CompilerParams(needs_layout_passes=False),
      scratch_types=[
          pltpu.VMEM((NCHUNK,), jnp.float32),
          pltpu.SMEM((KPAD,), jnp.int32),
          pltpu.SMEM((KPAD,), jnp.float32),
          pltpu.VMEM((8, CHUNK), jnp.float32),
          pltpu.VMEM((8, CHUNK), jnp.float32),
          pltpu.VMEM((CHUNK,), jnp.float32),
          pltpu.VMEM((P,), jnp.float32),
          pltpu.VMEM((P,), jnp.int32),
          pltpu.VMEM((KPAD,), jnp.float32),
          pltpu.VMEM((KPAD,), jnp.int32),
          pltpu.VMEM((NCHUNK // 16,), jnp.float32),
          pltpu.SemaphoreType.DMA,
      ],
  )
  xtail = jnp.pad(x[:, NFULL * CHUNK:], ((0, 0), (0, CHUNK - TAIL)),
                  constant_values=NEG_INF)
  vals3, idx3 = fn(x, xtail.reshape(R, 1, CHUNK), m.reshape(R, 1, NCHUNK))
  return vals3.reshape(R, KPAD), idx3.reshape(R, KPAD)


def _c1_body(x_ref, t_ref, c_ref):
  cb = pl.program_id(1)

  @pl.when(cb == 0)
  def _():
    c_ref[...] = jnp.zeros_like(c_ref)

  x = x_ref[...]
  t = t_ref[...]
  eq = (x == t).astype(jnp.float32)

  @pl.when(cb < 7)
  def _():
    c_ref[...] += jnp.sum(eq, axis=1).reshape(8, 1)

  @pl.when(cb == 7)
  def _():
    lane = lax.broadcasted_iota(jnp.int32, (8, _A_CB), 1)
    valid = lane < (N - 7 * _A_CB)
    c_ref[...] += jnp.sum(jnp.where(valid, eq, 0.0), axis=1).reshape(8, 1)


def _tie_count(x, t):
  return pl.pallas_call(
      _c1_body,
      grid=(R // 8, 8),
      in_specs=[
          pl.BlockSpec((8, _A_CB), lambda rb, cb: (rb, cb)),
          pl.BlockSpec((8, 1), lambda rb, cb: (rb, 0)),
      ],
      out_specs=pl.BlockSpec((8, 1), lambda rb, cb: (rb, 0)),
      out_shape=jax.ShapeDtypeStruct((R, 1), jnp.float32),
  )(x, t)


_C2_CB = 32768


def _c2_body(x_ref, sc_ref, o_ref):
  x = x_ref[...]
  t = sc_ref[:, 0:1]
  mt = sc_ref[:, 1:2]
  inv_s = sc_ref[:, 2:3]
  e = jnp.exp(x * INV_T - mt) * inv_s
  o_ref[...] = jnp.where(x >= t, e, 0.0)


def _probs(x, sc):
  return pl.pallas_call(
      _c2_body,
      grid=(R // 8, N // _C2_CB + 1),
      in_specs=[
          pl.BlockSpec((8, _C2_CB), lambda rb, cb: (rb, cb)),
          pl.BlockSpec((8, 4), lambda rb, cb: (rb, 0)),
      ],
      out_specs=pl.BlockSpec((8, _C2_CB), lambda rb, cb: (rb, cb)),
      out_shape=jax.ShapeDtypeStruct((R, N), jnp.float32),
  )(x, sc)


def kernel(logits):
  m = _chunk_maxes(logits)
  vals, idxs = _sc_topk(logits, m)
  t = vals[:, K - 1:K]
  mt = vals[:, 0:1] * INV_T
  c_total = _tie_count(logits, t)
  topk = vals[:, :K]
  e_top = jnp.exp(topk * INV_T - mt)
  c_in = jnp.sum((topk == t).astype(jnp.float32), axis=1, keepdims=True)
  s = jnp.sum(e_top, axis=1, keepdims=True) + (
      c_total - c_in) * jnp.exp(t * INV_T - mt)
  sc = jnp.concatenate([t, mt, 1.0 / s, jnp.zeros_like(s)], axis=1)
  probs = _probs(logits, sc)
  return probs, idxs[:, :K]


# speedup vs baseline: 144.6884x; 1.2385x over previous
import functools

import jax
import jax.numpy as jnp
from jax import lax
from jax.experimental import pallas as pl
from jax.experimental.pallas import tpu as pltpu
from jax.experimental.pallas import tpu_sc as plsc

R = 64
N = 1_000_000
K = 50
INV_T = 1.25
CHUNK = 1024
NCHUNK = 1024
NEG_INF = float("-inf")
BIG_I = 2**30
P = 1280
KPAD = 64
NFULL = N // CHUNK
TAIL = N - NFULL * CHUNK

_A_CB = 131072


def _a_body(x_ref, m_ref):
  cb = pl.program_id(1)

  @pl.when(cb < 7)
  def _():
    outs = []
    for ji in range(_A_CB // CHUNK):
      sub = x_ref[:, ji * CHUNK:(ji + 1) * CHUNK]
      outs.append(jnp.max(sub, axis=1).reshape(8, 1))
    m_ref[...] = jnp.concatenate(outs, axis=1)

  @pl.when(cb == 7)
  def _():
    lane = lax.broadcasted_iota(jnp.int32, (8, CHUNK), 1)
    outs = []
    for ji in range(_A_CB // CHUNK):
      sub = x_ref[:, ji * CHUNK:(ji + 1) * CHUNK]
      rem = N - (7 * _A_CB + ji * CHUNK)
      sub = jnp.where(lane < rem, sub, NEG_INF)
      outs.append(jnp.max(sub, axis=1).reshape(8, 1))
    m_ref[...] = jnp.concatenate(outs, axis=1)


def _chunk_maxes(x):
  return pl.pallas_call(
      _a_body,
      grid=(R // 8, 8),
      in_specs=[pl.BlockSpec((8, _A_CB), lambda rb, cb: (rb, cb))],
      out_specs=pl.BlockSpec((8, 128), lambda rb, cb: (rb, cb)),
      out_shape=jax.ShapeDtypeStruct((R, NCHUNK), jnp.float32),
  )(x)


def _sc_body(x_hbm, xtail_hbm, m_hbm, vals_hbm, idx_hbm,
             mv, cids, cmaxs, cbuf, cbuf2, tailbuf, bufv, bufi, topv, topi,
             supv, sem):
  info = plsc.get_sparse_core_info()
  nc = info.num_cores
  wid = lax.axis_index("s") * nc + lax.axis_index("c")
  iota = lax.iota(jnp.int32, 16)
  neg = jnp.full((16,), NEG_INF, jnp.float32)
  bigv = jnp.full((16,), BIG_I, jnp.int32)

  def vmax_s(v):
    m = v[0]
    for q in range(1, 16):
      m = jnp.maximum(m, v[q])
    return m

  def vmin_i(v):
    m = v[0]
    for q in range(1, 16):
      m = jnp.minimum(m, v[q])
    return m

  def nextbelow(x):
    xs = jnp.full((16,), x, jnp.float32)
    b = plsc.bitcast(xs, jnp.uint32)
    sgn = b >> jnp.uint32(31)
    key = b ^ jnp.where(sgn == jnp.uint32(1),
                        jnp.uint32(0xFFFFFFFF), jnp.uint32(0x80000000))
    key = key - jnp.uint32(1)
    sgn2 = key >> jnp.uint32(31)
    b2 = key ^ jnp.where(sgn2 == jnp.uint32(1),
                         jnp.uint32(0x80000000), jnp.uint32(0xFFFFFFFF))
    return plsc.bitcast(b2, jnp.float32)[0]

  def reselect(cnt, tdrop, tval):
    nsl = (cnt + 15) // 16
    def kbody(k, _):
      def sbody(i, carry):
        bv, bi, bp = carry
        p = iota + i * 16
        live = p < cnt
        v = jnp.where(live, bufv[pl.ds(i * 16, 16)], NEG_INF)
        g = jnp.where(live, bufi[pl.ds(i * 16, 16)], BIG_I)
        take = jnp.logical_or(v > bv, jnp.logical_and(v == bv, g < bi))
        return (jnp.where(take, v, bv), jnp.where(take, g, bi),
                jnp.where(take, p, bp))
      bv, bi, bp = lax.fori_loop(0, nsl, sbody, (neg, bigv, bigv))
      bestv = vmax_s(bv)
      besti = vmin_i(jnp.where(bv == bestv, bi, BIG_I))
      bpos = vmin_i(jnp.where(jnp.logical_and(bv == bestv, bi == besti),
                              bp, BIG_I))
      tb = (k // 16) * 16
      sel = (iota + tb) == k
      topv[pl.ds(tb, 16)] = jnp.where(sel, bestv, topv[pl.ds(tb, 16)])
      topi[pl.ds(tb, 16)] = jnp.where(sel, besti, topi[pl.ds(tb, 16)])
      cb = (bpos // 16) * 16
      bufv[pl.ds(cb, 16)] = jnp.where((iota + cb) == bpos, NEG_INF,
                                      bufv[pl.ds(cb, 16)])
      return 0
    lax.fori_loop(0, K, kbody, 0)

    v50 = topv[pl.ds(48, 16)][K - 1 - 48]

    def tiecnt(i, acc):
      p = iota + i * 16
      live = p < cnt
      v = jnp.where(live, bufv[pl.ds(i * 16, 16)], NEG_INF)
      return acc + plsc.all_reduce_population_count(v == v50)[0]
    ties = lax.fori_loop(0, nsl, tiecnt, jnp.int32(0))
    tdrop = jnp.where(v50 == tval, tdrop + ties, ties)

    def wbody(i, _):
      bufv[pl.ds(i * 16, 16)] = topv[pl.ds(i * 16, 16)]
      bufi[pl.ds(i * 16, 16)] = topi[pl.ds(i * 16, 16)]
      return 0
    lax.fori_loop(0, KPAD // 16, wbody, 0)
    return jnp.int32(K), nextbelow(v50), tdrop, v50

  def do_row(r):
    rb8 = pl.multiple_of((r // 8) * 8, 8)
    rsub = r - rb8
    pltpu.sync_copy(m_hbm.at[r, 0], mv)

    def prefill(i, _):
      bufv[pl.ds(i * 16, 16)] = neg
      bufi[pl.ds(i * 16, 16)] = bigv
      return 0
    lax.fori_loop(0, P // 16, prefill, 0)

    def prefill_top(i, _):
      topv[pl.ds(i * 16, 16)] = neg
      topi[pl.ds(i * 16, 16)] = bigv
      return 0
    lax.fori_loop(0, KPAD // 16, prefill_top, 0)

    def initsup(b, _):
      m16 = vmax_s(mv[pl.ds(b * 16, 16)])
      sb = (b // 16) * 16
      supv[pl.ds(sb, 16)] = jnp.where((iota + sb) == b, m16,
                                      supv[pl.ds(sb, 16)])
      return 0
    lax.fori_loop(0, NCHUNK // 16, initsup, 0)

    def kbody(k, _):
      def ssup(i, carry):
        bv, bb = carry
        v = supv[pl.ds(i * 16, 16)]
        g = iota + i * 16
        take = jnp.logical_or(v > bv, jnp.logical_and(v == bv, g < bb))
        return jnp.where(take, v, bv), jnp.where(take, g, bb)
      bv, bb = lax.fori_loop(0, NCHUNK // 256, ssup, (neg, bigv))
      bestv = vmax_s(bv)
      bblock = vmin_i(jnp.where(bv == bestv, bb, BIG_I))
      sl = mv[pl.ds(bblock * 16, 16)]
      lane = vmin_i(jnp.where(sl == bestv, iota, BIG_I))
      besti = bblock * 16 + lane
      cids[k] = besti
      cmaxs[k] = bestv
      sl2 = jnp.where(iota == lane, NEG_INF, sl)
      mv[pl.ds(bblock * 16, 16)] = sl2
      m2 = vmax_s(sl2)
      sb = (bblock // 16) * 16
      supv[pl.ds(sb, 16)] = jnp.where((iota + sb) == bblock, m2,
                                      supv[pl.ds(sb, 16)])
      return 0
    lax.fori_loop(0, K, kbody, 0)

    def filter_chunk(load_fn, lo, hi, cnt, t):
      def ibody(i, cnt):
        v = load_fn(i)
        g = iota + (lo + i * 16)
        pm = jnp.logical_and(g < hi, v > t)
        n = plsc.all_reduce_population_count(pm)[0]
        plsc.store_compressed(bufv.at[pl.ds(cnt, 16)], v, mask=pm)
        plsc.store_compressed(bufi.at[pl.ds(cnt, 16)], g, mask=pm)
        return cnt + n
      cnt = lax.fori_loop(0, CHUNK // 16, ibody, cnt, unroll=4)
      return cnt, t

    def start_fetch(j, buf):
      @pl.when(cids[j] != NFULL)
      def _():
        lo = cids[j] * CHUNK
        pltpu.make_async_copy(
            x_hbm.at[pl.ds(rb8, 8), pl.ds(lo, CHUNK)], buf, sem).start()

    def wait_fetch(j, buf):
      @pl.when(cids[j] != NFULL)
      def _():
        lo = cids[j] * CHUNK
        pltpu.make_async_copy(
            x_hbm.at[pl.ds(rb8, 8), pl.ds(lo, CHUNK)], buf, sem).wait()

    def process(j, buf, cnt, t, td, tv):
      cnt, t, td, tv = lax.cond(cnt > P - CHUNK,
                                lambda: reselect(cnt, td, tv),
                                lambda: (cnt, t, td, tv))
      cid = cids[j]
      lo = cid * CHUNK
      hi = jnp.minimum(lo + CHUNK, N)

      def full_path():
        return filter_chunk(lambda i: buf[rsub, pl.ds(i * 16, 16)],
                            lo, hi, cnt, t)

      def tail_path():
        pltpu.sync_copy(xtail_hbm.at[r, 0], tailbuf)
        return filter_chunk(lambda i: tailbuf[pl.ds(i * 16, 16)],
                            lo, hi, cnt, t)

      cnt, t = lax.cond(cid == NFULL, tail_path, full_path)
      return cnt, t, td, tv

    t0m = nextbelow(cmaxs[K - 1])

    start_fetch(0, cbuf)

    def pairbody(p, carry):
      cnt, t, td, tv = carry
      j0 = 2 * p
      wait_fetch(j0, cbuf)
      start_fetch(j0 + 1, cbuf2)
      cnt, t, td, tv = process(j0, cbuf, cnt, t, td, tv)
      wait_fetch(j0 + 1, cbuf2)

      @pl.when(p < K // 2 - 1)
      def _():
        start_fetch(j0 + 2, cbuf)

      return process(j0 + 1, cbuf2, cnt, t, td, tv)

    cnt_f, _, td_f, tv_f = lax.fori_loop(
        0, K // 2, pairbody,
        (jnp.int32(0), t0m, jnp.int32(0), jnp.float32(NEG_INF)))
    _, _, c_extra, _ = reselect(cnt_f, td_f, tv_f)
    topv[pl.ds(48, 16)] = jnp.where((iota + 48) == K,
                                    c_extra.astype(jnp.float32),
                                    topv[pl.ds(48, 16)])
    pltpu.sync_copy(topv, vals_hbm.at[r, 0])
    pltpu.sync_copy(topi, idx_hbm.at[r, 0])

  do_row(2 * wid)
  do_row(2 * wid + 1)


def _sc_topk(x, m):
  mesh = plsc.VectorSubcoreMesh(core_axis_name="c", subcore_axis_name="s")
  fn = pl.kernel(
      _sc_body,
      out_type=[jax.ShapeDtypeStruct((R, 1, KPAD), jnp.float32),
                jax.ShapeDtypeStruct((R, 1, KPAD), jnp.int32)],
      mesh=mesh,
      compiler_params=pltpu.CompilerParams(needs_layout_passes=False),
      scratch_types=[
          pltpu.VMEM((NCHUNK,), jnp.float32),
          pltpu.SMEM((KPAD,), jnp.int32),
          pltpu.SMEM((KPAD,), jnp.float32),
          pltpu.VMEM((8, CHUNK), jnp.float32),
          pltpu.VMEM((8, CHUNK), jnp.float32),
          pltpu.VMEM((CHUNK,), jnp.float32),
          pltpu.VMEM((P,), jnp.float32),
          pltpu.VMEM((P,), jnp.int32),
          pltpu.VMEM((KPAD,), jnp.float32),
          pltpu.VMEM((KPAD,), jnp.int32),
          pltpu.VMEM((NCHUNK // 16,), jnp.float32),
          pltpu.SemaphoreType.DMA,
      ],
  )
  xtail = jnp.pad(x[:, NFULL * CHUNK:], ((0, 0), (0, CHUNK - TAIL)),
                  constant_values=NEG_INF)
  vals3, idx3 = fn(x, xtail.reshape(R, 1, CHUNK), m.reshape(R, 1, NCHUNK))
  return vals3.reshape(R, KPAD), idx3.reshape(R, KPAD)


_C2_CB = 32768


def _c2_body(x_ref, sc_ref, o_ref):
  x = x_ref[...]
  t = sc_ref[:, 0:1]
  mt = sc_ref[:, 1:2]
  inv_s = sc_ref[:, 2:3]
  e = jnp.exp(x * INV_T - mt) * inv_s
  o_ref[...] = jnp.where(x >= t, e, 0.0)


def _probs(x, sc):
  return pl.pallas_call(
      _c2_body,
      grid=(R // 8, N // _C2_CB + 1),
      in_specs=[
          pl.BlockSpec((8, _C2_CB), lambda rb, cb: (rb, cb)),
          pl.BlockSpec((8, 4), lambda rb, cb: (rb, 0)),
      ],
      out_specs=pl.BlockSpec((8, _C2_CB), lambda rb, cb: (rb, cb)),
      out_shape=jax.ShapeDtypeStruct((R, N), jnp.float32),
  )(x, sc)


def kernel(logits):
  m = _chunk_maxes(logits)
  vals, idxs = _sc_topk(logits, m)
  t = vals[:, K - 1:K]
  mt = vals[:, 0:1] * INV_T
  c_extra = vals[:, K:K + 1]
  topk = vals[:, :K]
  e_top = jnp.exp(topk * INV_T - mt)
  s = jnp.sum(e_top, axis=1, keepdims=True) + c_extra * jnp.exp(t * INV_T - mt)
  sc = jnp.concatenate([t, mt, 1.0 / s, jnp.zeros_like(s)], axis=1)
  probs = _probs(logits, sc)
  return probs, idxs[:, :K]


# speedup vs baseline: 171.5201x; 1.1854x over previous
import functools

import jax
import jax.numpy as jnp
from jax import lax
from jax.experimental import pallas as pl
from jax.experimental.pallas import tpu as pltpu
from jax.experimental.pallas import tpu_sc as plsc

R = 64
N = 1_000_000
K = 50
INV_T = 1.25
CHUNK = 1024
NCHUNK = 1024
NEG_INF = float("-inf")
BIG_I = 2**30
P = 1280
KPAD = 64
NFULL = N // CHUNK
TAIL = N - NFULL * CHUNK

_A_CB = 131072


def _a_body(x_ref, m_ref):
  cb = pl.program_id(1)

  @pl.when(cb < 7)
  def _():
    outs = []
    for ji in range(_A_CB // CHUNK):
      sub = x_ref[:, ji * CHUNK:(ji + 1) * CHUNK]
      outs.append(jnp.max(sub, axis=1).reshape(16, 1))
    m_ref[...] = jnp.concatenate(outs, axis=1)

  @pl.when(cb == 7)
  def _():
    lane = lax.broadcasted_iota(jnp.int32, (16, CHUNK), 1)
    outs = []
    for ji in range(_A_CB // CHUNK):
      sub = x_ref[:, ji * CHUNK:(ji + 1) * CHUNK]
      rem = N - (7 * _A_CB + ji * CHUNK)
      sub = jnp.where(lane < rem, sub, NEG_INF)
      outs.append(jnp.max(sub, axis=1).reshape(16, 1))
    m_ref[...] = jnp.concatenate(outs, axis=1)


def _chunk_maxes(x):
  return pl.pallas_call(
      _a_body,
      grid=(R // 16, 8),
      in_specs=[pl.BlockSpec((16, _A_CB), lambda rb, cb: (rb, cb))],
      out_specs=pl.BlockSpec((16, 128), lambda rb, cb: (rb, cb)),
      out_shape=jax.ShapeDtypeStruct((R, NCHUNK), jnp.float32),
  )(x)


def _sc_body(x_hbm, xtail_hbm, m_hbm, vals_hbm, idx_hbm,
             mv, cids, cmaxs, cbuf, cbuf2, tailbuf, bufv, bufi, topv, topi,
             supv, sem):
  info = plsc.get_sparse_core_info()
  nc = info.num_cores
  wid = lax.axis_index("s") * nc + lax.axis_index("c")
  iota = lax.iota(jnp.int32, 16)
  neg = jnp.full((16,), NEG_INF, jnp.float32)
  bigv = jnp.full((16,), BIG_I, jnp.int32)

  def vmax_s(v):
    m = v[0]
    for q in range(1, 16):
      m = jnp.maximum(m, v[q])
    return m

  def vmin_i(v):
    m = v[0]
    for q in range(1, 16):
      m = jnp.minimum(m, v[q])
    return m

  def nextbelow(x):
    xs = jnp.full((16,), x, jnp.float32)
    b = plsc.bitcast(xs, jnp.uint32)
    sgn = b >> jnp.uint32(31)
    key = b ^ jnp.where(sgn == jnp.uint32(1),
                        jnp.uint32(0xFFFFFFFF), jnp.uint32(0x80000000))
    key = key - jnp.uint32(1)
    sgn2 = key >> jnp.uint32(31)
    b2 = key ^ jnp.where(sgn2 == jnp.uint32(1),
                         jnp.uint32(0x80000000), jnp.uint32(0xFFFFFFFF))
    return plsc.bitcast(b2, jnp.float32)[0]

  def reselect(cnt, tdrop, tval):
    nsl = (cnt + 15) // 16
    def kbody(k, _):
      def sbody(i, carry):
        bv, bi, bp = carry
        p = iota + i * 16
        live = p < cnt
        v = jnp.where(live, bufv[pl.ds(i * 16, 16)], NEG_INF)
        g = jnp.where(live, bufi[pl.ds(i * 16, 16)], BIG_I)
        take = jnp.logical_or(v > bv, jnp.logical_and(v == bv, g < bi))
        return (jnp.where(take, v, bv), jnp.where(take, g, bi),
                jnp.where(take, p, bp))
      bv, bi, bp = lax.fori_loop(0, nsl, sbody, (neg, bigv, bigv))
      bestv = vmax_s(bv)
      besti = vmin_i(jnp.where(bv == bestv, bi, BIG_I))
      bpos = vmin_i(jnp.where(jnp.logical_and(bv == bestv, bi == besti),
                              bp, BIG_I))
      tb = (k // 16) * 16
      sel = (iota + tb) == k
      topv[pl.ds(tb, 16)] = jnp.where(sel, bestv, topv[pl.ds(tb, 16)])
      topi[pl.ds(tb, 16)] = jnp.where(sel, besti, topi[pl.ds(tb, 16)])
      cb = (bpos // 16) * 16
      bufv[pl.ds(cb, 16)] = jnp.where((iota + cb) == bpos, NEG_INF,
                                      bufv[pl.ds(cb, 16)])
      return 0
    lax.fori_loop(0, K, kbody, 0)

    v50 = topv[pl.ds(48, 16)][K - 1 - 48]

    def tiecnt(i, acc):
      p = iota + i * 16
      live = p < cnt
      v = jnp.where(live, bufv[pl.ds(i * 16, 16)], NEG_INF)
      return acc + plsc.all_reduce_population_count(v == v50)[0]
    ties = lax.fori_loop(0, nsl, tiecnt, jnp.int32(0))
    tdrop = jnp.where(v50 == tval, tdrop + ties, ties)

    def wbody(i, _):
      bufv[pl.ds(i * 16, 16)] = topv[pl.ds(i * 16, 16)]
      bufi[pl.ds(i * 16, 16)] = topi[pl.ds(i * 16, 16)]
      return 0
    lax.fori_loop(0, KPAD // 16, wbody, 0)
    return jnp.int32(K), nextbelow(v50), tdrop, v50

  def do_row(r):
    rb8 = pl.multiple_of((r // 8) * 8, 8)
    rsub = r - rb8
    pltpu.sync_copy(m_hbm.at[r, 0], mv)

    def prefill(i, _):
      bufv[pl.ds(i * 16, 16)] = neg
      bufi[pl.ds(i * 16, 16)] = bigv
      return 0
    lax.fori_loop(0, P // 16, prefill, 0)

    def prefill_top(i, _):
      topv[pl.ds(i * 16, 16)] = neg
      topi[pl.ds(i * 16, 16)] = bigv
      return 0
    lax.fori_loop(0, KPAD // 16, prefill_top, 0)

    def initsup(b, _):
      m16 = vmax_s(mv[pl.ds(b * 16, 16)])
      sb = (b // 16) * 16
      supv[pl.ds(sb, 16)] = jnp.where((iota + sb) == b, m16,
                                      supv[pl.ds(sb, 16)])
      return 0
    lax.fori_loop(0, NCHUNK // 16, initsup, 0)

    def kbody(k, _):
      def ssup(i, carry):
        bv, bb = carry
        v = supv[pl.ds(i * 16, 16)]
        g = iota + i * 16
        take = jnp.logical_or(v > bv, jnp.logical_and(v == bv, g < bb))
        return jnp.where(take, v, bv), jnp.where(take, g, bb)
      bv, bb = lax.fori_loop(0, NCHUNK // 256, ssup, (neg, bigv))
      bestv = vmax_s(bv)
      bblock = vmin_i(jnp.where(bv == bestv, bb, BIG_I))
      sl = mv[pl.ds(bblock * 16, 16)]
      lane = vmin_i(jnp.where(sl == bestv, iota, BIG_I))
      besti = bblock * 16 + lane
      cids[k] = besti
      cmaxs[k] = bestv
      sl2 = jnp.where(iota == lane, NEG_INF, sl)
      mv[pl.ds(bblock * 16, 16)] = sl2
      m2 = vmax_s(sl2)
      sb = (bblock // 16) * 16
      supv[pl.ds(sb, 16)] = jnp.where((iota + sb) == bblock, m2,
                                      supv[pl.ds(sb, 16)])
      return 0
    lax.fori_loop(0, K, kbody, 0)

    def filter_chunk(load_fn, lo, hi, cnt, t):
      def ibody(i, cnt):
        v = load_fn(i)
        g = iota + (lo + i * 16)
        pm = jnp.logical_and(g < hi, v > t)
        n = plsc.all_reduce_population_count(pm)[0]
        plsc.store_compressed(bufv.at[pl.ds(cnt, 16)], v, mask=pm)
        plsc.store_compressed(bufi.at[pl.ds(cnt, 16)], g, mask=pm)
        return cnt + n
      cnt = lax.fori_loop(0, CHUNK // 16, ibody, cnt, unroll=4)
      return cnt, t

    def start_fetch(j, buf):
      @pl.when(cids[j] != NFULL)
      def _():
        lo = cids[j] * CHUNK
        pltpu.make_async_copy(
            x_hbm.at[pl.ds(rb8, 8), pl.ds(lo, CHUNK)], buf, sem).start()

    def wait_fetch(j, buf):
      @pl.when(cids[j] != NFULL)
      def _():
        lo = cids[j] * CHUNK
        pltpu.make_async_copy(
            x_hbm.at[pl.ds(rb8, 8), pl.ds(lo, CHUNK)], buf, sem).wait()

    def process(j, buf, cnt, t, td, tv):
      cnt, t, td, tv = lax.cond(cnt > P - CHUNK,
                                lambda: reselect(cnt, td, tv),
                                lambda: (cnt, t, td, tv))
      cid = cids[j]
      lo = cid * CHUNK
      hi = jnp.minimum(lo + CHUNK, N)

      def full_path():
        return filter_chunk(lambda i: buf[rsub, pl.ds(i * 16, 16)],
                            lo, hi, cnt, t)

      def tail_path():
        pltpu.sync_copy(xtail_hbm.at[r, 0], tailbuf)
        return filter_chunk(lambda i: tailbuf[pl.ds(i * 16, 16)],
                            lo, hi, cnt, t)

      cnt, t = lax.cond(cid == NFULL, tail_path, full_path)
      return cnt, t, td, tv

    t0m = nextbelow(cmaxs[K - 1])

    start_fetch(0, cbuf)

    def pairbody(p, carry):
      cnt, t, td, tv = carry
      j0 = 2 * p
      wait_fetch(j0, cbuf)
      start_fetch(j0 + 1, cbuf2)
      cnt, t, td, tv = process(j0, cbuf, cnt, t, td, tv)
      wait_fetch(j0 + 1, cbuf2)

      @pl.when(p < K // 2 - 1)
      def _():
        start_fetch(j0 + 2, cbuf)

      return process(j0 + 1, cbuf2, cnt, t, td, tv)

    cnt_f, _, td_f, tv_f = lax.fori_loop(
        0, K // 2, pairbody,
        (jnp.int32(0), t0m, jnp.int32(0), jnp.float32(NEG_INF)))
    _, _, c_extra, _ = reselect(cnt_f, td_f, tv_f)
    topv[pl.ds(48, 16)] = jnp.where((iota + 48) == K,
                                    c_extra.astype(jnp.float32),
                                    topv[pl.ds(48, 16)])
    pltpu.sync_copy(topv, vals_hbm.at[r, 0])
    pltpu.sync_copy(topi, idx_hbm.at[r, 0])

  do_row(2 * wid)
  do_row(2 * wid + 1)


def _sc_topk(x, m):
  mesh = plsc.VectorSubcoreMesh(core_axis_name="c", subcore_axis_name="s")
  fn = pl.kernel(
      _sc_body,
      out_type=[jax.ShapeDtypeStruct((R, 1, KPAD), jnp.float32),
                jax.ShapeDtypeStruct((R, 1, KPAD), jnp.int32)],
      mesh=mesh,
      compiler_params=pltpu.CompilerParams(needs_layout_passes=False),
      scratch_types=[
          pltpu.VMEM((NCHUNK,), jnp.float32),
          pltpu.SMEM((KPAD,), jnp.int32),
          pltpu.SMEM((KPAD,), jnp.float32),
          pltpu.VMEM((8, CHUNK), jnp.float32),
          pltpu.VMEM((8, CHUNK), jnp.float32),
          pltpu.VMEM((CHUNK,), jnp.float32),
          pltpu.VMEM((P,), jnp.float32),
          pltpu.VMEM((P,), jnp.int32),
          pltpu.VMEM((KPAD,), jnp.float32),
          pltpu.VMEM((KPAD,), jnp.int32),
          pltpu.VMEM((NCHUNK // 16,), jnp.float32),
          pltpu.SemaphoreType.DMA,
      ],
  )
  xtail = jnp.pad(x[:, NFULL * CHUNK:], ((0, 0), (0, CHUNK - TAIL)),
                  constant_values=NEG_INF)
  vals3, idx3 = fn(x, xtail.reshape(R, 1, CHUNK), m.reshape(R, 1, NCHUNK))
  return vals3.reshape(R, KPAD), idx3.reshape(R, KPAD)


_C2_CB = 32768


def _c2_body(x_ref, sc_ref, o_ref):
  x = x_ref[...]
  t = sc_ref[:, 0:1]
  mt = sc_ref[:, 1:2]
  inv_s = sc_ref[:, 2:3]
  e = jnp.exp(x * INV_T - mt) * inv_s
  o_ref[...] = jnp.where(x >= t, e, 0.0)


def _probs(x, sc):
  return pl.pallas_call(
      _c2_body,
      grid=(R // 16, N // _C2_CB + 1),
      in_specs=[
          pl.BlockSpec((16, _C2_CB), lambda rb, cb: (rb, cb)),
          pl.BlockSpec((16, 4), lambda rb, cb: (rb, 0)),
      ],
      out_specs=pl.BlockSpec((16, _C2_CB), lambda rb, cb: (rb, cb)),
      out_shape=jax.ShapeDtypeStruct((R, N), jnp.float32),
  )(x, sc)


def kernel(logits):
  m = _chunk_maxes(logits)
  vals, idxs = _sc_topk(logits, m)
  t = vals[:, K - 1:K]
  mt = vals[:, 0:1] * INV_T
  c_extra = vals[:, K:K + 1]
  topk = vals[:, :K]
  e_top = jnp.exp(topk * INV_T - mt)
  s = jnp.sum(e_top, axis=1, keepdims=True) + c_extra * jnp.exp(t * INV_T - mt)
  sc = jnp.concatenate([t, mt, 1.0 / s, jnp.zeros_like(s)], axis=1)
  probs = _probs(logits, sc)
  return probs, idxs[:, :K]
